# baseline (device time: 207380 ns/iter reference)
import jax
import jax.numpy as jnp
from jax import lax
from jax.experimental import pallas as pl
from jax.experimental.pallas import tpu as pltpu

N_DEV = 16
E_LOCAL = 4
N_EXPERTS = N_DEV * E_LOCAL


def kernel(x, router_W, route_idx, expert_W, shared_W):
    T, D = x.shape
    H = shared_W.shape[1]

    def body(x_ref, router_ref, idx_ref, expert_ref, shared_ref, out_ref,
             comm_ref, send_sems, recv_sems):
        my = lax.axis_index("i")
        left = lax.rem(my + N_DEV - 1, N_DEV)
        right = lax.rem(my + 1, N_DEV)

        barrier_sem = pltpu.get_barrier_semaphore()
        for nbr in (left, right):
            pl.semaphore_signal(
                barrier_sem, inc=1,
                device_id=(nbr,), device_id_type=pl.DeviceIdType.MESH,
            )
        pl.semaphore_wait(barrier_sem, 2)

        xf = x_ref[:, :]
        scores = jnp.dot(xf, router_ref[:, :],
                         preferred_element_type=jnp.float32)
        m = jnp.max(scores, axis=-1, keepdims=True)
        ex = jnp.exp(scores - m)
        probs = ex / jnp.sum(ex, axis=-1, keepdims=True)
        g = idx_ref[:, :]
        onehot = lax.broadcasted_iota(jnp.int32, (T, N_EXPERTS), 1) == g
        p = jnp.sum(jnp.where(onehot, probs, 0.0), axis=-1,
                    keepdims=True)

        xb = xf.astype(jnp.bfloat16)

        out_ref[:, :] = jnp.dot(xb, shared_ref[:, :].astype(jnp.bfloat16),
                                preferred_element_type=jnp.float32)

        comm_ref[0] = (
            expert_ref[:, :, :].reshape(E_LOCAL * D, H).astype(jnp.bfloat16)
        )

        def compute(slot):
            src = lax.rem(my + N_DEV - slot, N_DEV)
            base = src * E_LOCAL
            cols = [
                jnp.where(g == base + e, xb, jnp.bfloat16(0.0))
                for e in range(E_LOCAL)
            ]
            xg = jnp.concatenate(cols, axis=1)
            t = jnp.dot(xg, comm_ref[slot],
                        preferred_element_type=jnp.float32)
            in_shard = (g >= base) & (g < base + E_LOCAL)
            coef = jnp.where(in_shard, p, 0.0)
            out_ref[:, :] += coef * t

        for h in range(N_DEV - 1):
            rdma = pltpu.make_async_remote_copy(
                src_ref=comm_ref.at[h],
                dst_ref=comm_ref.at[h + 1],
                send_sem=send_sems.at[h],
                recv_sem=recv_sems.at[h],
                device_id=(right,),
                device_id_type=pl.DeviceIdType.MESH,
            )
            rdma.start()
            compute(h)
            rdma.wait()
        compute(N_DEV - 1)

    return pl.pallas_call(
        body,
        out_shape=jax.ShapeDtypeStruct((T, H), jnp.float32),
        in_specs=[pl.BlockSpec(memory_space=pltpu.VMEM)] * 5,
        out_specs=pl.BlockSpec(memory_space=pltpu.VMEM),
        scratch_shapes=[
            pltpu.VMEM((N_DEV, E_LOCAL * D, H), jnp.bfloat16),
            pltpu.SemaphoreType.DMA((N_DEV - 1,)),
            pltpu.SemaphoreType.DMA((N_DEV - 1,)),
        ],
        compiler_params=pltpu.CompilerParams(collective_id=0),
    )(x, router_W, route_idx, expert_W, shared_W)


# device time: 128240 ns/iter; 1.6171x vs baseline; 1.6171x over previous
import jax
import jax.numpy as jnp
from jax import lax
from jax.experimental import pallas as pl
from jax.experimental.pallas import tpu as pltpu

N_DEV = 16
E_LOCAL = 4
N_EXPERTS = N_DEV * E_LOCAL
R_HOPS = 8
L_HOPS = 7


def kernel(x, router_W, route_idx, expert_W, shared_W):
    T, D = x.shape
    H = shared_W.shape[1]

    def body(x_ref, router_ref, idx_ref, expert_ref, shared_ref, out_ref,
             comm_r, comm_l, send_r, recv_r, send_l, recv_l):
        my = lax.axis_index("i")
        left = lax.rem(my + N_DEV - 1, N_DEV)
        right = lax.rem(my + 1, N_DEV)

        barrier_sem = pltpu.get_barrier_semaphore()
        for nbr in (left, right):
            pl.semaphore_signal(
                barrier_sem, inc=1,
                device_id=(nbr,), device_id_type=pl.DeviceIdType.MESH,
            )
        pl.semaphore_wait(barrier_sem, 2)

        xf = x_ref[:, :]
        scores = jnp.dot(xf, router_ref[:, :],
                         preferred_element_type=jnp.float32)
        m = jnp.max(scores, axis=-1, keepdims=True)
        ex = jnp.exp(scores - m)
        probs = ex / jnp.sum(ex, axis=-1, keepdims=True)
        g = idx_ref[:, :]
        onehot = lax.broadcasted_iota(jnp.int32, (T, N_EXPERTS), 1) == g
        p = jnp.sum(jnp.where(onehot, probs, 0.0), axis=-1,
                    keepdims=True)

        xb = xf.astype(jnp.bfloat16)

        out_ref[:, :] = jnp.dot(xb, shared_ref[:, :].astype(jnp.bfloat16),
                                preferred_element_type=jnp.float32)

        own = expert_ref[:, :, :].reshape(E_LOCAL * D, H).astype(jnp.bfloat16)
        comm_r[0] = own
        comm_l[0] = own

        def compute(buf, slot, off):
            src = lax.rem(my + N_DEV + off, N_DEV)
            base = src * E_LOCAL
            cols = [
                jnp.where(g == base + e, xb, jnp.bfloat16(0.0))
                for e in range(E_LOCAL)
            ]
            xg = jnp.concatenate(cols, axis=1)
            t = jnp.dot(xg, buf[slot],
                        preferred_element_type=jnp.float32)
            in_shard = (g >= base) & (g < base + E_LOCAL)
            coef = jnp.where(in_shard, p, 0.0)
            out_ref[:, :] += coef * t

        for h in range(R_HOPS):
            rdma_r = pltpu.make_async_remote_copy(
                src_ref=comm_r.at[h],
                dst_ref=comm_r.at[h + 1],
                send_sem=send_r.at[h],
                recv_sem=recv_r.at[h],
                device_id=(right,),
                device_id_type=pl.DeviceIdType.MESH,
            )
            rdma_r.start()
            if h < L_HOPS:
                rdma_l = pltpu.make_async_remote_copy(
                    src_ref=comm_l.at[h],
                    dst_ref=comm_l.at[h + 1],
                    send_sem=send_l.at[h],
                    recv_sem=recv_l.at[h],
                    device_id=(left,),
                    device_id_type=pl.DeviceIdType.MESH,
                )
                rdma_l.start()
            compute(comm_r, h, -h)
            if h >= 1:
                compute(comm_l, h, h)
            rdma_r.wait()
            if h < L_HOPS:
                rdma_l.wait()
        compute(comm_r, R_HOPS, -R_HOPS)

    return pl.pallas_call(
        body,
        out_shape=jax.ShapeDtypeStruct((T, H), jnp.float32),
        in_specs=[pl.BlockSpec(memory_space=pltpu.VMEM)] * 5,
        out_specs=pl.BlockSpec(memory_space=pltpu.VMEM),
        scratch_shapes=[
            pltpu.VMEM((R_HOPS + 1, E_LOCAL * D, H), jnp.bfloat16),
            pltpu.VMEM((L_HOPS + 1, E_LOCAL * D, H), jnp.bfloat16),
            pltpu.SemaphoreType.DMA((R_HOPS,)),
            pltpu.SemaphoreType.DMA((R_HOPS,)),
            pltpu.SemaphoreType.DMA((L_HOPS,)),
            pltpu.SemaphoreType.DMA((L_HOPS,)),
        ],
        compiler_params=pltpu.CompilerParams(collective_id=0),
    )(x, router_W, route_idx, expert_W, shared_W)
